# baseline (speedup 1.0000x reference)
import functools

import jax
import jax.numpy as jnp
from jax import lax
from jax.experimental import pallas as pl
from jax.experimental.pallas import tpu as pltpu, tpu_sc as plsc

ROWS = 4096
COLS = 26
VOCAB = 1000
PLANE = COLS * VOCAB
TOTAL = ROWS * PLANE
NUM_WORKERS = 32
PLANES_PER_WORKER = ROWS // NUM_WORKERS
POS_PER_WORKER = PLANES_PER_WORKER * COLS
L = 16
K = 128
CHUNKS = POS_PER_WORKER // K
MSET_F = 1_331_200
MSET_N = TOTAL // MSET_F


def _memset_body(o_ref, zbuf, sem):
    zbuf[...] = jnp.zeros((MSET_F,), jnp.float32)

    def fire(i, carry):
        pltpu.async_copy(zbuf, o_ref.at[pl.ds(i * MSET_F, MSET_F)], sem)
        return carry

    lax.fori_loop(0, MSET_N, fire, 0)

    def drain(i, carry):
        pltpu.make_async_copy(zbuf, o_ref.at[pl.ds(0, MSET_F)], sem).wait()
        return carry

    lax.fori_loop(0, MSET_N, drain, 0)


_memset = functools.partial(
    pl.kernel,
    mesh=pltpu.create_tensorcore_mesh("core"),
    scratch_types=[
        pltpu.VMEM((MSET_F,), jnp.float32),
        pltpu.SemaphoreType.DMA,
    ],
)(_memset_body)


def _scatter_body(x_hbm, out_ref, xs_v, idx_v, ones_v, sem):
    wid = lax.axis_index("c") * 16 + lax.axis_index("s")
    base = wid * PLANES_PER_WORKER

    pltpu.sync_copy(x_hbm.at[pl.ds(base, PLANES_PER_WORKER)], xs_v)

    iota = lax.iota(jnp.int32, L)
    for m in range(K // L):
        ones_v[pl.ds(m * L, L)] = jnp.full((L,), 1.0, jnp.float32)

    def fill_idx(c, carry):
        for m in range(K // L):
            r = c * K + m * L + iota
            poff = r // COLS
            j = r - poff * COLS
            cols = plsc.load_gather(xs_v, [poff, j])
            idx_v[c, pl.ds(m * L, L)] = (
                (base + poff) * PLANE + j * VOCAB + cols)
        return carry

    lax.fori_loop(0, CHUNKS, fill_idx, 0)

    def fire(c, carry):
        pltpu.async_copy(ones_v, out_ref.at[idx_v.at[c]], sem)
        return carry

    lax.fori_loop(0, CHUNKS, fire, 0)

    def drain(c, carry):
        pltpu.make_async_copy(ones_v, out_ref.at[idx_v.at[0]], sem).wait()
        return carry

    lax.fori_loop(0, CHUNKS, drain, 0)


_scatter = functools.partial(
    pl.kernel,
    mesh=plsc.VectorSubcoreMesh(core_axis_name="c", subcore_axis_name="s"),
    compiler_params=pltpu.CompilerParams(
        use_tc_tiling_on_sc=False, needs_layout_passes=False),
    scratch_types=[
        pltpu.VMEM((PLANES_PER_WORKER, COLS), jnp.int32),
        pltpu.VMEM((CHUNKS, K), jnp.int32),
        pltpu.VMEM((K,), jnp.float32),
        pltpu.SemaphoreType.DMA,
    ],
)(_scatter_body)


def _finish_body(i_ref, o_ref, tbuf, sem):
    cp_in = pltpu.make_async_copy(i_ref.at[pl.ds(0, 128)], tbuf, sem)
    cp_in.start()
    cp_in.wait()
    cp_out = pltpu.make_async_copy(tbuf, o_ref.at[pl.ds(0, 128)], sem)
    cp_out.start()
    cp_out.wait()


_finish = pl.pallas_call(
    _finish_body,
    in_specs=[pl.BlockSpec(memory_space=pl.ANY)],
    out_specs=pl.BlockSpec(memory_space=pl.ANY),
    out_shape=jax.ShapeDtypeStruct((TOTAL,), jnp.float32),
    input_output_aliases={0: 0},
    scratch_shapes=[
        pltpu.VMEM((128,), jnp.float32),
        pltpu.SemaphoreType.DMA,
    ],
)


def kernel(x):
    out = jax.empty_ref(jax.ShapeDtypeStruct((TOTAL,), jnp.float32))
    _memset(out)
    _scatter(x, out)
    return _finish(jax.ref.freeze(out)).reshape(ROWS, COLS, VOCAB)


# speedup vs baseline: 1.0575x; 1.0575x over previous
import functools

import jax
import jax.numpy as jnp
from jax import lax
from jax.experimental import pallas as pl
from jax.experimental.pallas import tpu as pltpu, tpu_sc as plsc

ROWS = 4096
COLS = 26
VOCAB = 1000
PLANE = COLS * VOCAB
NUM_WORKERS = 32
PLANES_PER_WORKER = ROWS // NUM_WORKERS
L = 16
P = 1
GROUPS = PLANES_PER_WORKER // P
NBATCH = (P * COLS + L - 1) // L


def _batch_consts():
    iota = lax.iota(jnp.int32, L)
    out = []
    for k in range(NBATCH):
        r = iota + k * L
        mask = (r < P * COLS) if (k + 1) * L > P * COLS else None
        rc = jnp.minimum(r, P * COLS - 1)
        out.append((rc // COLS, rc % COLS, mask))
    return out


NBUF = 4


def _body(x_hbm, zeros_hbm, out_hbm, buf0, buf1, buf2, buf3, idx_v, sav,
          sem0, sem1, sem2, sem3):
    wid = lax.axis_index("c") * 16 + lax.axis_index("s")
    base = wid * PLANES_PER_WORKER

    bufs = (buf0, buf1, buf2, buf3)
    sems = (sem0, sem1, sem2, sem3)

    for b in range(NBUF):
        pltpu.async_copy(zeros_hbm, bufs[b], sems[b])

    pltpu.sync_copy(x_hbm.at[pl.ds(base, PLANES_PER_WORKER)], idx_v)

    consts = _batch_consts()
    ones = jnp.full((L,), 1.0, jnp.float32)
    zeros_v = jnp.zeros((L,), jnp.float32)
    zeros_i = jnp.zeros((L,), jnp.int32)

    for i in range(NBUF * NBATCH):
        sav[i, :] = zeros_i

    def step(g, carry):
        for b in range(NBUF):
            buf, sem = bufs[b], sems[b]
            first_plane = (NBUF * g + b) * P
            pltpu.make_async_copy(zeros_hbm, buf, sem).wait()
            for k, (poff, rows, mask) in enumerate(consts):
                slot = b * NBATCH + k
                plsc.store_scatter(buf, [sav[slot, :]], zeros_v, mask=mask)
                cols = plsc.load_gather(idx_v, [first_plane + poff, rows],
                                        mask=mask)
                pos = poff * PLANE + rows * VOCAB + cols
                if mask is not None:
                    pos = jnp.where(mask, pos, 0)
                plsc.store_scatter(buf, [pos], ones, mask=mask)
                sav[slot, :] = pos
            pltpu.async_copy(
                buf, out_hbm.at[pl.ds((base + first_plane) * PLANE,
                                      P * PLANE)], sem)
        return carry

    lax.fori_loop(0, GROUPS // NBUF, step, 0)

    for b in range(NBUF):
        pltpu.make_async_copy(zeros_hbm, bufs[b], sems[b]).wait()


_onehot_sc = functools.partial(
    pl.kernel,
    out_type=jax.ShapeDtypeStruct((ROWS * PLANE,), jnp.float32),
    mesh=plsc.VectorSubcoreMesh(core_axis_name="c", subcore_axis_name="s"),
    compiler_params=pltpu.CompilerParams(
        use_tc_tiling_on_sc=False, needs_layout_passes=False),
    scratch_types=[
        pltpu.VMEM((P * PLANE,), jnp.float32),
        pltpu.VMEM((P * PLANE,), jnp.float32),
        pltpu.VMEM((P * PLANE,), jnp.float32),
        pltpu.VMEM((P * PLANE,), jnp.float32),
        pltpu.VMEM((PLANES_PER_WORKER, COLS), jnp.int32),
        pltpu.VMEM((4 * NBATCH, L), jnp.int32),
        pltpu.SemaphoreType.DMA,
        pltpu.SemaphoreType.DMA,
        pltpu.SemaphoreType.DMA,
        pltpu.SemaphoreType.DMA,
    ],
)(_body)


def kernel(x):
    zeros = jnp.zeros((P * PLANE,), jnp.float32)
    return _onehot_sc(x, zeros).reshape(ROWS, COLS, VOCAB)


# speedup vs baseline: 1.0725x; 1.0142x over previous
import functools

import jax
import jax.numpy as jnp
from jax import lax
from jax.experimental import pallas as pl
from jax.experimental.pallas import tpu as pltpu, tpu_sc as plsc

ROWS = 4096
COLS = 26
VOCAB = 1000
PLANE = COLS * VOCAB
NUM_WORKERS = 32
PLANES_PER_WORKER = ROWS // NUM_WORKERS
L = 16
P = 2
GROUPS = PLANES_PER_WORKER // P
NBATCH = (P * COLS + L - 1) // L


def _batch_consts():
    iota = lax.iota(jnp.int32, L)
    out = []
    for k in range(NBATCH):
        r = iota + k * L
        mask = (r < P * COLS) if (k + 1) * L > P * COLS else None
        rc = jnp.minimum(r, P * COLS - 1)
        out.append((rc // COLS, rc % COLS, mask))
    return out


def _body(x_hbm, zeros_hbm, out_hbm, buf0, buf1, idx_v, sav, sem0, sem1):
    wid = lax.axis_index("c") * 16 + lax.axis_index("s")
    base = wid * PLANES_PER_WORKER

    bufs = (buf0, buf1)
    sems = (sem0, sem1)

    pltpu.async_copy(zeros_hbm, buf0, sem0)
    pltpu.async_copy(zeros_hbm, buf1, sem1)

    pltpu.sync_copy(x_hbm.at[pl.ds(base, PLANES_PER_WORKER)], idx_v)

    consts = _batch_consts()
    ones = jnp.full((L,), 1.0, jnp.float32)
    zeros_v = jnp.zeros((L,), jnp.float32)
    zeros_i = jnp.zeros((L,), jnp.int32)

    for i in range(2 * NBATCH):
        sav[i, :] = zeros_i

    def step(g, carry):
        for b in range(2):
            buf, sem = bufs[b], sems[b]
            first_plane = (2 * g + b) * P
            pltpu.make_async_copy(zeros_hbm, buf, sem).wait()
            for k, (poff, rows, mask) in enumerate(consts):
                slot = b * NBATCH + k
                plsc.store_scatter(buf, [sav[slot, :]], zeros_v, mask=mask)
                cols = plsc.load_gather(idx_v, [first_plane + poff, rows],
                                        mask=mask)
                pos = poff * PLANE + rows * VOCAB + cols
                if mask is not None:
                    pos = jnp.where(mask, pos, 0)
                plsc.store_scatter(buf, [pos], ones, mask=mask)
                sav[slot, :] = pos
            pltpu.async_copy(
                buf, out_hbm.at[pl.ds((base + first_plane) * PLANE,
                                      P * PLANE)], sem)
        return carry

    lax.fori_loop(0, GROUPS // 2, step, 0)

    pltpu.make_async_copy(zeros_hbm, buf0, sem0).wait()
    pltpu.make_async_copy(zeros_hbm, buf1, sem1).wait()


_onehot_sc = functools.partial(
    pl.kernel,
    out_type=jax.ShapeDtypeStruct((ROWS * PLANE,), jnp.float32),
    mesh=plsc.VectorSubcoreMesh(core_axis_name="c", subcore_axis_name="s"),
    compiler_params=pltpu.CompilerParams(
        use_tc_tiling_on_sc=False, needs_layout_passes=False),
    scratch_types=[
        pltpu.VMEM((P * PLANE,), jnp.float32),
        pltpu.VMEM((P * PLANE,), jnp.float32),
        pltpu.VMEM((PLANES_PER_WORKER, COLS), jnp.int32),
        pltpu.VMEM((2 * NBATCH, L), jnp.int32),
        pltpu.SemaphoreType.DMA,
        pltpu.SemaphoreType.DMA,
    ],
)(_body)


def kernel(x):
    zeros = jnp.zeros((P * PLANE,), jnp.float32)
    return _onehot_sc(x, zeros).reshape(ROWS, COLS, VOCAB)
